# baseline (speedup 1.0000x reference)
import functools

import jax
import jax.numpy as jnp
from jax import lax
from jax.experimental import pallas as pl
from jax.experimental.pallas import tpu as pltpu
from jax.experimental.pallas import tpu_sc as plsc

_NC = 2
_NS = 16
_NW = _NC * _NS
_L = 16
_CHUNK = 128


def _tc_transpose_body(x_ref, o_ref):
    xt = x_ref[...].T
    o_ref[:, :64] = xt[:128]
    o_ref[:, 64:] = xt[128:]


@functools.lru_cache(maxsize=None)
def _make_tc_transpose(V, D):
    n_blocks = pl.cdiv(V, 256)
    return pl.pallas_call(
        _tc_transpose_body,
        grid=(n_blocks,),
        in_specs=[pl.BlockSpec((D, 256), lambda j: (0, j))],
        out_specs=pl.BlockSpec((128, 2 * D), lambda j: (j, 0)),
        out_shape=jax.ShapeDtypeStruct((n_blocks * 128, 2 * D), jnp.float32),
    )


@functools.lru_cache(maxsize=None)
def _make_sc_gather(V, D, B):
    b_per_w = B // _NW
    n_chunks = b_per_w // _CHUNK
    mesh = plsc.VectorSubcoreMesh(core_axis_name="c", subcore_axis_name="s")

    @functools.partial(
        pl.kernel,
        mesh=mesh,
        out_type=jax.ShapeDtypeStruct((B // 2, 2 * D), jnp.float32),
        scratch_types=[
            pltpu.VMEM((n_chunks, _CHUNK), jnp.int32),
            pltpu.VMEM((n_chunks, _CHUNK), jnp.int32),
            pltpu.VMEM((b_per_w, 2 * D), jnp.float32),
            pltpu.VMEM((b_per_w // 2, 2 * D), jnp.float32),
            pltpu.SemaphoreType.DMA,
        ],
        compiler_params=pltpu.CompilerParams(use_tc_tiling_on_sc=True,
                                             needs_layout_passes=False),
    )
    def gather_kernel(pairs_hbm, idx_hbm, out_hbm,
                      idx_v, pid_v, rows128_v, rows_v, sem):
        w = lax.axis_index("s") * _NC + lax.axis_index("c")
        pltpu.sync_copy(idx_hbm.at[w], idx_v)
        lanes = lax.iota(jnp.int32, _L)
        for j in range(n_chunks):
            for g in range(_CHUNK // _L):
                sl = pl.ds(g * _L, _L)
                iv = idx_v[j, sl]
                pid_v[j, sl] = ((iv >> 8) << 7) + (iv & 127)
        copies = [
            pltpu.async_copy(
                pairs_hbm.at[pid_v.at[j]],
                rows128_v.at[pl.ds(j * _CHUNK, _CHUNK)],
                sem,
            )
            for j in range(n_chunks)
        ]
        for c in copies:
            c.wait()
        for j in range(n_chunks):
            for g in range(_CHUNK // _L):
                row = j * _CHUNK + g * _L + lanes
                iv = idx_v[j, pl.ds(g * _L, _L)]
                half = ((iv >> 7) & 1) * D
                p_vec = row >> 1
                qbase = (row & 1) * D

                def col_body(i, _, row=row, half=half, p_vec=p_vec,
                             qbase=qbase):
                    for u in range(4):
                        c = i * 4 + u
                        val = plsc.load_gather(rows128_v, [row, half + c])
                        plsc.store_scatter(rows_v, [p_vec, qbase + c], val)
                    return 0

                lax.fori_loop(0, D // 4, col_body, 0)
        pltpu.sync_copy(rows_v,
                        out_hbm.at[pl.ds(w * (b_per_w // 2), b_per_w // 2)])

    return gather_kernel


def kernel(inputs, dim, indices):
    V, D = inputs.shape
    B = indices.shape[0]
    idx = (indices + jnp.asarray(dim, dtype=indices.dtype)).astype(jnp.int32)
    idx = idx.reshape(_NW, B // _NW // _CHUNK, _CHUNK)
    pairs = _make_tc_transpose(V, D)(inputs.T)
    out2 = _make_sc_gather(V, D, B)(pairs, idx)
    return out2.reshape(B, D)


# speedup vs baseline: 3.3997x; 3.3997x over previous
import functools

import jax
import jax.numpy as jnp
from jax import lax
from jax.experimental import pallas as pl
from jax.experimental.pallas import tpu as pltpu
from jax.experimental.pallas import tpu_sc as plsc

_NC = 2
_NS = 16
_NW = _NC * _NS
_L = 16
_CHUNK = 128


_TCB = 2048


def _tc_transpose_body(x_ref, o_ref):
    x = x_ref[...]
    eye = (lax.broadcasted_iota(jnp.int32, (64, 64), 0)
           == lax.broadcasted_iota(jnp.int32, (64, 64), 1)).astype(jnp.float32)
    xt = lax.dot_general(x, eye, (((0,), (0,)), ((), ())),
                         precision=lax.Precision.HIGHEST)
    for m in range(_TCB // 256):
        o_ref[m * 128:(m + 1) * 128, :64] = xt[m * 256:m * 256 + 128]
        o_ref[m * 128:(m + 1) * 128, 64:] = xt[m * 256 + 128:(m + 1) * 256]


@functools.lru_cache(maxsize=None)
def _make_tc_transpose(V, D):
    n_blocks = pl.cdiv(V, _TCB)
    return pl.pallas_call(
        _tc_transpose_body,
        grid=(n_blocks,),
        in_specs=[pl.BlockSpec((D, _TCB), lambda j: (0, j))],
        out_specs=pl.BlockSpec((_TCB // 2, 2 * D), lambda j: (j, 0)),
        out_shape=jax.ShapeDtypeStruct((n_blocks * _TCB // 2, 2 * D),
                                       jnp.float32),
    )


@functools.lru_cache(maxsize=None)
def _make_sc_gather(V, D, B):
    b_per_w = B // _NW
    n_chunks = b_per_w // _CHUNK
    mesh = plsc.VectorSubcoreMesh(core_axis_name="c", subcore_axis_name="s")

    @functools.partial(
        pl.kernel,
        mesh=mesh,
        out_type=jax.ShapeDtypeStruct((B // 2, 2 * D), jnp.float32),
        scratch_types=[
            pltpu.VMEM((n_chunks, _CHUNK), jnp.int32),
            pltpu.VMEM((n_chunks, _CHUNK), jnp.int32),
            pltpu.VMEM((b_per_w, 2 * D), jnp.float32),
            pltpu.VMEM((b_per_w // 2, 2 * D), jnp.float32),
            pltpu.SemaphoreType.DMA,
        ],
        compiler_params=pltpu.CompilerParams(use_tc_tiling_on_sc=True,
                                             needs_layout_passes=False),
    )
    def gather_kernel(pairs_hbm, idx_hbm, out_hbm,
                      idx_v, pid_v, rows128_v, rows_v, sem):
        w = lax.axis_index("s") * _NC + lax.axis_index("c")
        pltpu.sync_copy(idx_hbm.at[w], idx_v)
        lanes = lax.iota(jnp.int32, _L)
        for j in range(n_chunks):
            for g in range(_CHUNK // _L):
                sl = pl.ds(g * _L, _L)
                iv = idx_v[j, sl]
                pid_v[j, sl] = ((iv >> 8) << 7) + (iv & 127)
        copies = [
            pltpu.async_copy(
                pairs_hbm.at[pid_v.at[j]],
                rows128_v.at[pl.ds(j * _CHUNK, _CHUNK)],
                sem,
            )
            for j in range(n_chunks)
        ]
        for j in range(n_chunks):
            copies[j].wait()
            for g in range(_CHUNK // _L):
                row = j * _CHUNK + g * _L + lanes
                iv = idx_v[j, pl.ds(g * _L, _L)]
                half = ((iv >> 7) & 1) * D
                p_vec = row >> 1
                qbase = (row & 1) * D

                def col_body(i, _, row=row, half=half, p_vec=p_vec,
                             qbase=qbase):
                    for u in range(8):
                        c = i * 8 + u
                        val = plsc.load_gather(rows128_v, [row, half + c])
                        plsc.store_scatter(rows_v, [p_vec, qbase + c], val)
                    return 0

                lax.fori_loop(0, D // 8, col_body, 0)
        pltpu.sync_copy(rows_v,
                        out_hbm.at[pl.ds(w * (b_per_w // 2), b_per_w // 2)])

    return gather_kernel


def kernel(inputs, dim, indices):
    V, D = inputs.shape
    B = indices.shape[0]
    idx = (indices + jnp.asarray(dim, dtype=indices.dtype)).astype(jnp.int32)
    idx = idx.reshape(_NW, B // _NW // _CHUNK, _CHUNK)
    pairs = _make_tc_transpose(V, D)(inputs.T)
    out2 = _make_sc_gather(V, D, B)(pairs, idx)
    return out2.reshape(B, D)


# speedup vs baseline: 5.1804x; 1.5238x over previous
import functools

import jax
import jax.numpy as jnp
from jax import lax
from jax.experimental import pallas as pl
from jax.experimental.pallas import tpu as pltpu
from jax.experimental.pallas import tpu_sc as plsc

_NC = 2
_NS = 16
_NW = _NC * _NS
_L = 16
_CHUNK = 128


_TCB = 4096


def _tc_transpose_body(x_ref, o_ref):
    x = x_ref[...]
    eye = (lax.broadcasted_iota(jnp.int32, (64, 64), 0)
           == lax.broadcasted_iota(jnp.int32, (64, 64), 1)).astype(jnp.bfloat16)
    hi = x.astype(jnp.bfloat16)
    r1 = x - hi.astype(jnp.float32)
    mid = r1.astype(jnp.bfloat16)
    lo = (r1 - mid.astype(jnp.float32)).astype(jnp.bfloat16)
    dims = (((0,), (0,)), ((), ()))
    xt = lax.dot_general(hi, eye, dims, preferred_element_type=jnp.float32)
    xt = xt + lax.dot_general(mid, eye, dims,
                              preferred_element_type=jnp.float32)
    xt = xt + lax.dot_general(lo, eye, dims,
                              preferred_element_type=jnp.float32)
    for m in range(_TCB // 256):
        o_ref[m * 128:(m + 1) * 128, :64] = xt[m * 256:m * 256 + 128]
        o_ref[m * 128:(m + 1) * 128, 64:] = xt[m * 256 + 128:(m + 1) * 256]


@functools.lru_cache(maxsize=None)
def _make_tc_transpose(V, D):
    n_blocks = pl.cdiv(V, _TCB)
    return pl.pallas_call(
        _tc_transpose_body,
        grid=(n_blocks,),
        in_specs=[pl.BlockSpec((D, _TCB), lambda j: (0, j))],
        out_specs=pl.BlockSpec((_TCB // 2, 2 * D), lambda j: (j, 0)),
        out_shape=jax.ShapeDtypeStruct((n_blocks * _TCB // 2, 2 * D),
                                       jnp.float32),
    )


@functools.lru_cache(maxsize=None)
def _make_sc_gather(V, D, B):
    b_per_w = B // _NW
    n_chunks = b_per_w // _CHUNK
    mesh = plsc.VectorSubcoreMesh(core_axis_name="c", subcore_axis_name="s")

    @functools.partial(
        pl.kernel,
        mesh=mesh,
        out_type=jax.ShapeDtypeStruct((B // 2, 2 * D), jnp.float32),
        scratch_types=[
            pltpu.VMEM((n_chunks, _CHUNK), jnp.int32),
            pltpu.VMEM((n_chunks, _CHUNK), jnp.int32),
            pltpu.VMEM((b_per_w, 2 * D), jnp.float32),
            pltpu.VMEM((b_per_w // 2, 2 * D), jnp.float32),
            pltpu.SemaphoreType.DMA,
        ],
        compiler_params=pltpu.CompilerParams(use_tc_tiling_on_sc=True,
                                             needs_layout_passes=False),
    )
    def gather_kernel(pairs_hbm, idx_hbm, out_hbm,
                      idx_v, pid_v, rows128_v, rows_v, sem):
        w = lax.axis_index("s") * _NC + lax.axis_index("c")
        pltpu.sync_copy(idx_hbm.at[w], idx_v)
        lanes = lax.iota(jnp.int32, _L)
        for j in range(n_chunks):
            for g in range(_CHUNK // _L):
                sl = pl.ds(g * _L, _L)
                iv = idx_v[j, sl]
                pid_v[j, sl] = ((iv >> 8) << 7) + (iv & 127)
        copies = [
            pltpu.async_copy(
                pairs_hbm.at[pid_v.at[j]],
                rows128_v.at[pl.ds(j * _CHUNK, _CHUNK)],
                sem,
            )
            for j in range(n_chunks)
        ]
        for j in range(n_chunks):
            copies[j].wait()
            for g in range(_CHUNK // _L):
                row = j * _CHUNK + g * _L + lanes
                iv = idx_v[j, pl.ds(g * _L, _L)]
                half = ((iv >> 7) & 1) * D
                p_vec = row >> 1
                qbase = (row & 1) * D

                def col_body(i, _, row=row, half=half, p_vec=p_vec,
                             qbase=qbase):
                    for u in range(8):
                        c = i * 8 + u
                        val = plsc.load_gather(rows128_v, [row, half + c])
                        plsc.store_scatter(rows_v, [p_vec, qbase + c], val)
                    return 0

                lax.fori_loop(0, D // 8, col_body, 0)
        pltpu.sync_copy(rows_v,
                        out_hbm.at[pl.ds(w * (b_per_w // 2), b_per_w // 2)])

    return gather_kernel


def kernel(inputs, dim, indices):
    V, D = inputs.shape
    B = indices.shape[0]
    idx = (indices + jnp.asarray(dim, dtype=indices.dtype)).astype(jnp.int32)
    idx = idx.reshape(_NW, B // _NW // _CHUNK, _CHUNK)
    pairs = _make_tc_transpose(V, D)(inputs.T)
    out2 = _make_sc_gather(V, D, B)(pairs, idx)
    return out2.reshape(B, D)


# speedup vs baseline: 7.6509x; 1.4769x over previous
import functools

import jax
import jax.numpy as jnp
from jax import lax
from jax.experimental import pallas as pl
from jax.experimental.pallas import tpu as pltpu
from jax.experimental.pallas import tpu_sc as plsc

_NC = 2
_NS = 16
_NW = _NC * _NS
_L = 16
_CHUNK = 128


_TCB = 8192


def _tc_transpose_body(x_ref, o_ref):
    x = x_ref[...]
    eye = (lax.broadcasted_iota(jnp.int32, (256, 256), 0)
           == lax.broadcasted_iota(jnp.int32, (256, 256), 1)
           ).astype(jnp.bfloat16)
    hi = x.astype(jnp.bfloat16)
    r1 = x - hi.astype(jnp.float32)
    mid = r1.astype(jnp.bfloat16)
    lo = (r1 - mid.astype(jnp.float32)).astype(jnp.bfloat16)
    dims = (((0,), (0,)), ((), ()))
    for q in range(_TCB // 4096):
        parts = []
        for term in (hi, mid, lo):
            parts.append(jnp.concatenate(
                [term[:, q * 4096 + a * 1024:q * 4096 + (a + 1) * 1024]
                 for a in range(4)], axis=0))
        xt4 = lax.dot_general(parts[0], eye, dims,
                              preferred_element_type=jnp.float32)
        xt4 = xt4 + lax.dot_general(parts[1], eye, dims,
                                    preferred_element_type=jnp.float32)
        xt4 = xt4 + lax.dot_general(parts[2], eye, dims,
                                    preferred_element_type=jnp.float32)
        o_ref[q * 2048:q * 2048 + 1024, :] = xt4[:, :128]
        o_ref[q * 2048 + 1024:(q + 1) * 2048, :] = xt4[:, 128:]


@functools.lru_cache(maxsize=None)
def _make_tc_transpose(V, D):
    n_blocks = pl.cdiv(V, _TCB)
    return pl.pallas_call(
        _tc_transpose_body,
        grid=(n_blocks,),
        in_specs=[pl.BlockSpec((D, _TCB), lambda j: (0, j))],
        out_specs=pl.BlockSpec((_TCB // 2, 2 * D), lambda j: (j, 0)),
        out_shape=jax.ShapeDtypeStruct((n_blocks * _TCB // 2, 2 * D),
                                       jnp.float32),
    )


@functools.lru_cache(maxsize=None)
def _make_sc_gather(V, D, B):
    b_per_w = B // _NW
    n_chunks = b_per_w // _CHUNK
    mesh = plsc.VectorSubcoreMesh(core_axis_name="c", subcore_axis_name="s")

    @functools.partial(
        pl.kernel,
        mesh=mesh,
        out_type=jax.ShapeDtypeStruct((B // 2, 2 * D), jnp.float32),
        scratch_types=[
            pltpu.VMEM((n_chunks, _CHUNK), jnp.int32),
            pltpu.VMEM((n_chunks, _CHUNK), jnp.int32),
            pltpu.VMEM((b_per_w, 2 * D), jnp.float32),
            pltpu.VMEM((b_per_w // 2, 2 * D), jnp.float32),
            pltpu.SemaphoreType.DMA,
        ],
        compiler_params=pltpu.CompilerParams(use_tc_tiling_on_sc=True,
                                             needs_layout_passes=False),
    )
    def gather_kernel(pairs_hbm, idx_hbm, out_hbm,
                      idx_v, pid_v, rows128_v, rows_v, sem):
        w = lax.axis_index("s") * _NC + lax.axis_index("c")
        pltpu.sync_copy(idx_hbm.at[w], idx_v)
        lanes = lax.iota(jnp.int32, _L)
        for j in range(n_chunks):
            for g in range(_CHUNK // _L):
                sl = pl.ds(g * _L, _L)
                iv = idx_v[j, sl]
                pid_v[j, sl] = (((iv >> 12) << 11) + (((iv >> 11) & 1) << 10)
                                + (iv & 1023))
        copies = [
            pltpu.async_copy(
                pairs_hbm.at[pid_v.at[j]],
                rows128_v.at[pl.ds(j * _CHUNK, _CHUNK)],
                sem,
            )
            for j in range(n_chunks)
        ]
        for j in range(n_chunks):
            copies[j].wait()
            for g in range(_CHUNK // _L):
                row = j * _CHUNK + g * _L + lanes
                iv = idx_v[j, pl.ds(g * _L, _L)]
                half = ((iv >> 10) & 1) * D
                p_vec = row >> 1
                qbase = (row & 1) * D

                def col_body(i, _, row=row, half=half, p_vec=p_vec,
                             qbase=qbase):
                    for u in range(8):
                        c = i * 8 + u
                        val = plsc.load_gather(rows128_v, [row, half + c])
                        plsc.store_scatter(rows_v, [p_vec, qbase + c], val)
                    return 0

                lax.fori_loop(0, D // 8, col_body, 0)
        pltpu.sync_copy(rows_v,
                        out_hbm.at[pl.ds(w * (b_per_w // 2), b_per_w // 2)])

    return gather_kernel


def kernel(inputs, dim, indices):
    V, D = inputs.shape
    B = indices.shape[0]
    idx = (indices + jnp.asarray(dim, dtype=indices.dtype)).astype(jnp.int32)
    idx = idx.reshape(_NW, B // _NW // _CHUNK, _CHUNK)
    pairs = _make_tc_transpose(V, D)(inputs.T)
    out2 = _make_sc_gather(V, D, B)(pairs, idx)
    return out2.reshape(B, D)


# speedup vs baseline: 9.6029x; 1.2551x over previous
import functools

import jax
import jax.numpy as jnp
from jax import lax
from jax.experimental import pallas as pl
from jax.experimental.pallas import tpu as pltpu
from jax.experimental.pallas import tpu_sc as plsc

_NC = 2
_NS = 16
_NW = _NC * _NS
_L = 16
_CHUNK = 128


_TCB = 16384


def _tc_transpose_body(x_ref, o_ref):
    x = x_ref[...]
    eye = (lax.broadcasted_iota(jnp.int32, (256, 256), 0)
           == lax.broadcasted_iota(jnp.int32, (256, 256), 1)
           ).astype(jnp.bfloat16)
    hi = x.astype(jnp.bfloat16)
    r1 = x - hi.astype(jnp.float32)
    mid = r1.astype(jnp.bfloat16)
    lo = (r1 - mid.astype(jnp.float32)).astype(jnp.bfloat16)
    dims = (((0,), (0,)), ((), ()))
    for q in range(_TCB // 4096):
        parts = []
        for term in (hi, mid, lo):
            parts.append(jnp.concatenate(
                [term[:, q * 4096 + a * 1024:q * 4096 + (a + 1) * 1024]
                 for a in range(4)], axis=0))
        xt4 = lax.dot_general(parts[0], eye, dims,
                              preferred_element_type=jnp.float32)
        xt4 = xt4 + lax.dot_general(parts[1], eye, dims,
                                    preferred_element_type=jnp.float32)
        xt4 = xt4 + lax.dot_general(parts[2], eye, dims,
                                    preferred_element_type=jnp.float32)
        o_ref[q * 2048:q * 2048 + 1024, :] = xt4[:, :128]
        o_ref[q * 2048 + 1024:(q + 1) * 2048, :] = xt4[:, 128:]


@functools.lru_cache(maxsize=None)
def _make_tc_transpose(V, D):
    n_blocks = pl.cdiv(V, _TCB)
    return pl.pallas_call(
        _tc_transpose_body,
        grid=(n_blocks,),
        in_specs=[pl.BlockSpec((D, _TCB), lambda j: (0, j))],
        out_specs=pl.BlockSpec((_TCB // 2, 2 * D), lambda j: (j, 0)),
        out_shape=jax.ShapeDtypeStruct((n_blocks * _TCB // 2, 2 * D),
                                       jnp.float32),
    )


@functools.lru_cache(maxsize=None)
def _make_sc_gather(V, D, B):
    b_per_w = B // _NW
    n_chunks = b_per_w // _CHUNK
    mesh = plsc.VectorSubcoreMesh(core_axis_name="c", subcore_axis_name="s")

    @functools.partial(
        pl.kernel,
        mesh=mesh,
        out_type=jax.ShapeDtypeStruct((D, B), jnp.float32),
        scratch_types=[
            pltpu.VMEM((n_chunks, _CHUNK), jnp.int32),
            pltpu.VMEM((n_chunks, _CHUNK), jnp.int32),
            pltpu.VMEM((b_per_w, 2 * D), jnp.float32),
            pltpu.VMEM((D, b_per_w), jnp.float32),
            pltpu.SemaphoreType.DMA,
        ],
        compiler_params=pltpu.CompilerParams(use_tc_tiling_on_sc=True,
                                             needs_layout_passes=False),
    )
    def gather_kernel(pairs_hbm, idx_hbm, out_hbm,
                      idx_v, pid_v, rows128_v, rows_v, sem):
        w = lax.axis_index("s") * _NC + lax.axis_index("c")
        pltpu.sync_copy(idx_hbm.at[w], idx_v)
        lanes = lax.iota(jnp.int32, _L)
        for j in range(n_chunks):
            for g in range(_CHUNK // _L):
                sl = pl.ds(g * _L, _L)
                iv = idx_v[j, sl]
                pid_v[j, sl] = (((iv >> 12) << 11) + (((iv >> 11) & 1) << 10)
                                + (iv & 1023))
        copies = [
            pltpu.async_copy(
                pairs_hbm.at[pid_v.at[j]],
                rows128_v.at[pl.ds(j * _CHUNK, _CHUNK)],
                sem,
            )
            for j in range(n_chunks)
        ]
        for j in range(n_chunks):
            copies[j].wait()
            for g in range(_CHUNK // _L):
                row = j * _CHUNK + g * _L + lanes
                iv = idx_v[j, pl.ds(g * _L, _L)]
                half = ((iv >> 10) & 1) * D

                def col_body(i, _, row=row, half=half):
                    for u in range(8):
                        c = i * 8 + u
                        val = plsc.load_gather(rows128_v, [row, half + c])
                        cv = jnp.full((_L,), c, jnp.int32)
                        plsc.store_scatter(rows_v, [cv, row], val)
                    return 0

                lax.fori_loop(0, D // 8, col_body, 0)
        pltpu.sync_copy(rows_v, out_hbm.at[:, pl.ds(w * b_per_w, b_per_w)])

    return gather_kernel


def kernel(inputs, dim, indices):
    V, D = inputs.shape
    B = indices.shape[0]
    idx = (indices + jnp.asarray(dim, dtype=indices.dtype)).astype(jnp.int32)
    idx = idx.reshape(_NW, B // _NW // _CHUNK, _CHUNK)
    pairs = _make_tc_transpose(V, D)(inputs.T)
    outT = _make_sc_gather(V, D, B)(pairs, idx)
    return outT.T


# speedup vs baseline: 10.3058x; 1.0732x over previous
import functools

import jax
import jax.numpy as jnp
from jax import lax
from jax.experimental import pallas as pl
from jax.experimental.pallas import tpu as pltpu
from jax.experimental.pallas import tpu_sc as plsc

_NC = 2
_NS = 16
_NW = _NC * _NS
_L = 16
_CHUNK = 128


_TCB = 32768


def _tc_transpose_body(x_ref, o_ref):
    x = x_ref[...]
    eye = (lax.broadcasted_iota(jnp.int32, (256, 256), 0)
           == lax.broadcasted_iota(jnp.int32, (256, 256), 1)
           ).astype(jnp.bfloat16)
    hi = x.astype(jnp.bfloat16)
    r1 = x - hi.astype(jnp.float32)
    mid = r1.astype(jnp.bfloat16)
    lo = (r1 - mid.astype(jnp.float32)).astype(jnp.bfloat16)
    dims = (((0,), (0,)), ((), ()))
    for q in range(_TCB // 4096):
        parts = []
        for term in (hi, mid, lo):
            parts.append(jnp.concatenate(
                [term[:, q * 4096 + a * 1024:q * 4096 + (a + 1) * 1024]
                 for a in range(4)], axis=0))
        xt4 = lax.dot_general(parts[0], eye, dims,
                              preferred_element_type=jnp.float32)
        xt4 = xt4 + lax.dot_general(parts[1], eye, dims,
                                    preferred_element_type=jnp.float32)
        xt4 = xt4 + lax.dot_general(parts[2], eye, dims,
                                    preferred_element_type=jnp.float32)
        o_ref[q * 2048:q * 2048 + 1024, :] = xt4[:, :128]
        o_ref[q * 2048 + 1024:(q + 1) * 2048, :] = xt4[:, 128:]


@functools.lru_cache(maxsize=None)
def _make_tc_transpose(V, D):
    n_blocks = pl.cdiv(V, _TCB)
    return pl.pallas_call(
        _tc_transpose_body,
        grid=(n_blocks,),
        in_specs=[pl.BlockSpec((D, _TCB), lambda j: (0, j))],
        out_specs=pl.BlockSpec((_TCB // 2, 2 * D), lambda j: (j, 0)),
        out_shape=jax.ShapeDtypeStruct((n_blocks * _TCB // 2, 2 * D),
                                       jnp.float32),
    )


@functools.lru_cache(maxsize=None)
def _make_sc_gather(V, D, B):
    b_per_w = B // _NW
    n_chunks = b_per_w // _CHUNK
    mesh = plsc.VectorSubcoreMesh(core_axis_name="c", subcore_axis_name="s")

    @functools.partial(
        pl.kernel,
        mesh=mesh,
        out_type=jax.ShapeDtypeStruct((D, B), jnp.float32),
        scratch_types=[
            pltpu.VMEM((n_chunks, _CHUNK), jnp.int32),
            pltpu.VMEM((n_chunks, _CHUNK), jnp.int32),
            pltpu.VMEM((b_per_w, 2 * D), jnp.float32),
            pltpu.VMEM((D, b_per_w), jnp.float32),
            pltpu.SemaphoreType.DMA,
        ],
        compiler_params=pltpu.CompilerParams(use_tc_tiling_on_sc=True,
                                             needs_layout_passes=False),
    )
    def gather_kernel(pairs_hbm, idx_hbm, out_hbm,
                      idx_v, pid_v, rows128_v, rows_v, sem):
        w = lax.axis_index("s") * _NC + lax.axis_index("c")
        pltpu.sync_copy(idx_hbm.at[w], idx_v)
        lanes = lax.iota(jnp.int32, _L)
        for j in range(n_chunks):
            for g in range(_CHUNK // _L):
                sl = pl.ds(g * _L, _L)
                iv = idx_v[j, sl]
                pid_v[j, sl] = (((iv >> 12) << 11) + (((iv >> 11) & 1) << 10)
                                + (iv & 1023))
        copies = [
            pltpu.async_copy(
                pairs_hbm.at[pid_v.at[j]],
                rows128_v.at[pl.ds(j * _CHUNK, _CHUNK)],
                sem,
            )
            for j in range(n_chunks)
        ]
        for j in range(n_chunks):
            copies[j].wait()
            for g in range(_CHUNK // _L):
                row = j * _CHUNK + g * _L + lanes
                iv = idx_v[j, pl.ds(g * _L, _L)]
                half = ((iv >> 10) & 1) * D

                for c in range(D):
                    val = plsc.load_gather(rows128_v, [row, half + c])
                    cv = jnp.full((_L,), c, jnp.int32)
                    plsc.store_scatter(rows_v, [cv, row], val)
        pltpu.sync_copy(rows_v, out_hbm.at[:, pl.ds(w * b_per_w, b_per_w)])

    return gather_kernel


def kernel(inputs, dim, indices):
    V, D = inputs.shape
    B = indices.shape[0]
    idx = (indices + jnp.asarray(dim, dtype=indices.dtype)).astype(jnp.int32)
    idx = idx.reshape(_NW, B // _NW // _CHUNK, _CHUNK)
    pairs = _make_tc_transpose(V, D)(inputs.T)
    outT = _make_sc_gather(V, D, B)(pairs, idx)
    return outT.T
